# baseline (speedup 1.0000x reference)
import functools

import jax
import jax.numpy as jnp
import numpy as np
from jax import lax
from jax.experimental import pallas as pl
from jax.experimental.pallas import tpu as pltpu
from jax.experimental.pallas import tpu_sc as plsc

_VOCAB = 1000000
_DEPTH = 64
_LENGTH = 512
_BATCH = 1024

_LANES = 16


def _pos_encoding_np(length, depth):
    pos = np.arange(length)[:, None]
    i = np.arange(depth)[None, :]
    angle_rates = 1.0 / np.power(10000, 2 * (i // 2) / np.float32(depth))
    angle_rads = pos * angle_rates
    angle_rads[:, 0::2] = np.sin(angle_rads[:, 0::2])
    angle_rads[:, 1::2] = np.cos(angle_rads[:, 1::2])
    return angle_rads.astype(np.float32)


def _make_sc_kernel(n_rows, depth, length):
    info = plsc.get_sparse_core_info()
    nc, ns = info.num_cores, info.num_subcores
    nw = nc * ns
    per_w = n_rows // nw
    ch = length
    n_ch = per_w // ch
    mesh = plsc.VectorSubcoreMesh(core_axis_name="c", subcore_axis_name="s")

    @functools.partial(
        pl.kernel,
        out_type=jax.ShapeDtypeStruct((n_rows, depth), jnp.float32),
        mesh=mesh,
        scratch_types=[
            pltpu.VMEM((ch,), jnp.int32),
            pltpu.VMEM((ch, depth), jnp.float32),
            pltpu.VMEM((ch, depth), jnp.float32),
            pltpu.SemaphoreType.DMA,
        ],
        compiler_params=pltpu.CompilerParams(use_tc_tiling_on_sc=False),
    )
    def k(x_hbm, table_hbm, pe_hbm, out_hbm, idx_v, rows_v, pe_v, sem):
        wid = lax.axis_index("s") * nc + lax.axis_index("c")
        base_w = wid * per_w
        pltpu.sync_copy(pe_hbm, pe_v)

        def chunk_body(s, carry):
            base = base_w + s * ch
            pltpu.sync_copy(x_hbm.at[pl.ds(base, ch)], idx_v)
            pltpu.async_copy(table_hbm.at[idx_v], rows_v, sem).wait()

            def row_body(r, c2):
                for c in range(depth // _LANES):
                    sl = pl.ds(c * _LANES, _LANES)
                    rows_v[r, sl] = rows_v[r, sl] + pe_v[r, sl]
                return c2

            lax.fori_loop(0, ch, row_body, 0)
            pltpu.sync_copy(rows_v, out_hbm.at[pl.ds(base, ch)])
            return carry

        lax.fori_loop(0, n_ch, chunk_body, 0)

    return k


def kernel(x, table):
    pe = jnp.asarray(_pos_encoding_np(_LENGTH, _DEPTH))
    xf = x.reshape(-1).astype(jnp.int32)
    k = _make_sc_kernel(xf.shape[0], _DEPTH, _LENGTH)
    out = k(xf, table, pe)
    return out.reshape(_BATCH, _LENGTH, _DEPTH)
